# Initial kernel scaffold; baseline (speedup 1.0000x reference)
import jax
import jax.numpy as jnp
from jax.experimental import pallas as pl
from jax.experimental.pallas import tpu as pltpu

B, N, S1, K1, S2, K2 = 16, 2048, 512, 32, 128, 64
F32 = jnp.float32


def _mm(a, b):
    return jnp.dot(a, b, preferred_element_type=F32)


def _pairwise_d2(xyz, xyzT, n_rows, n_cols):
    acc = None
    for c in range(3):
        df = xyz[:n_rows, c:c + 1] - xyzT[c:c + 1, :n_cols]
        sq = df * df
        acc = sq if acc is None else acc + sq
    return acc


def _sa_kernel(xyz_ref, xyzT_ref,
               w0p_ref, b0_ref, w1_ref, b1_ref, w2_ref, b2_ref,
               w20a_ref, w20b_ref, b20_ref, w21_ref, b21_ref, w22_ref, b22_ref,
               w30a_ref, w30b_ref, b30_ref, w31_ref, b31_ref, w32_ref, b32_ref,
               out_ref, d2_ref, g1_ref, d22_ref, g2_ref):
    xyz = xyz_ref[0]
    xyzT = xyzT_ref[0]

    d2_ref[...] = _pairwise_d2(xyz, xyzT, S1, N)
    iota1 = jax.lax.broadcasted_iota(jnp.int32, (S1, N), 1)
    new_xyz1 = xyz[:S1, :]

    def step1(j, carry):
        d2 = d2_ref[...]
        rowmin = jnp.min(d2, axis=1, keepdims=True)
        amin = jnp.min(jnp.where(d2 == rowmin, iota1, N), axis=1, keepdims=True)
        oh_b = iota1 == amin
        oh = oh_b.astype(F32)
        sel = _mm(oh, xyz)
        g1_ref[j] = jnp.concatenate([sel - new_xyz1, sel], axis=1)
        d2_ref[...] = jnp.where(oh_b, jnp.float32(jnp.inf), d2)
        return carry

    jax.lax.fori_loop(0, K1, step1, 0)

    l1f = None
    CH1 = 8
    for c0 in range(0, K1, CH1):
        x = g1_ref[c0:c0 + CH1].reshape(CH1 * S1, 16)
        h = jnp.maximum(_mm(x, w0p_ref[...]) + b0_ref[...], 0.0)
        h = jnp.maximum(_mm(h, w1_ref[...]) + b1_ref[...], 0.0)
        h = jnp.maximum(_mm(h, w2_ref[...]) + b2_ref[...], 0.0)
        for j in range(CH1):
            hj = h[j * S1:(j + 1) * S1]
            l1f = hj if l1f is None else jnp.maximum(l1f, hj)

    d22_ref[...] = _pairwise_d2(xyz, xyzT, S2, S1)
    iota2 = jax.lax.broadcasted_iota(jnp.int32, (S2, S1), 1)
    a2 = _mm(xyz[:S1, :], w20a_ref[...]) + _mm(l1f, w20b_ref[...]) + b20_ref[...]
    c2 = _mm(xyz[:S2, :], w20a_ref[...])

    def step2(j, carry):
        d2 = d22_ref[...]
        rowmin = jnp.min(d2, axis=1, keepdims=True)
        amin = jnp.min(jnp.where(d2 == rowmin, iota2, S1), axis=1, keepdims=True)
        oh_b = iota2 == amin
        oh = oh_b.astype(F32)
        sel = _mm(oh, a2)
        g2_ref[j] = jnp.maximum(sel - c2, 0.0)
        d22_ref[...] = jnp.where(oh_b, jnp.float32(jnp.inf), d2)
        return carry

    jax.lax.fori_loop(0, K2, step2, 0)

    l2f = None
    CH2 = 16
    for c0 in range(0, K2, CH2):
        x = g2_ref[c0:c0 + CH2].reshape(CH2 * S2, 128)
        h = jnp.maximum(_mm(x, w21_ref[...]) + b21_ref[...], 0.0)
        h = jnp.maximum(_mm(h, w22_ref[...]) + b22_ref[...], 0.0)
        for j in range(CH2):
            hj = h[j * S2:(j + 1) * S2]
            l2f = hj if l2f is None else jnp.maximum(l2f, hj)

    h = jnp.maximum(_mm(xyz[:S2, :], w30a_ref[...]) + _mm(l2f, w30b_ref[...])
                    + b30_ref[...], 0.0)
    h = jnp.maximum(_mm(h, w31_ref[...]) + b31_ref[...], 0.0)
    h = jnp.maximum(_mm(h, w32_ref[...]) + b32_ref[...], 0.0)
    out_ref[...] = jnp.max(h, axis=0, keepdims=True)


def _head_kernel(gf_ref, fc1w_ref, fc1b_ref, bn1g_ref, bn1b_ref,
                 fc2w_ref, fc2b_ref, bn2g_ref, bn2b_ref,
                 fc3w_ref, fc3b_ref, out_ref):
    def bn_relu(x, g, b):
        m = jnp.mean(x, axis=0, keepdims=True)
        v = jnp.mean((x - m) ** 2, axis=0, keepdims=True)
        return jnp.maximum(g * (x - m) / jnp.sqrt(v + 1e-5) + b, 0.0)

    h = _mm(gf_ref[...], fc1w_ref[...]) + fc1b_ref[...]
    h = bn_relu(h, bn1g_ref[...], bn1b_ref[...])
    h = _mm(h, fc2w_ref[...]) + fc2b_ref[...]
    h = bn_relu(h, bn2g_ref[...], bn2b_ref[...])
    s = _mm(h, fc3w_ref[...]) + fc3b_ref[...]
    s = s - jnp.max(s, axis=1, keepdims=True)
    out_ref[...] = s - jnp.log(jnp.sum(jnp.exp(s), axis=1, keepdims=True))


def kernel(pts, sa1_w0, sa1_b0, sa1_w1, sa1_b1, sa1_w2, sa1_b2,
           sa2_w0, sa2_b0, sa2_w1, sa2_b1, sa2_w2, sa2_b2,
           sa3_w0, sa3_b0, sa3_w1, sa3_b1, sa3_w2, sa3_b2,
           fc1_w, fc1_b, bn1_g, bn1_b, fc2_w, fc2_b, bn2_g, bn2_b,
           fc3_w, fc3_b):
    xyz_pad = jnp.concatenate(
        [pts, jnp.zeros((B, N, 5), F32)], axis=-1)
    xyzT = jnp.transpose(xyz_pad, (0, 2, 1))

    w0p = jnp.zeros((16, 64), F32)
    w0p = w0p.at[0:3].set(sa1_w0[0:3]).at[8:11].set(sa1_w0[3:6])
    w20a = jnp.zeros((8, 128), F32).at[0:3].set(sa2_w0[0:3])
    w20b = sa2_w0[3:]
    w30a = jnp.zeros((8, 256), F32).at[0:3].set(sa3_w0[0:3])
    w30b = sa3_w0[3:]

    def row(v):
        return v[None, :]

    weights = (w0p, row(sa1_b0), sa1_w1, row(sa1_b1), sa1_w2, row(sa1_b2),
               w20a, w20b, row(sa2_b0), sa2_w1, row(sa2_b1), sa2_w2, row(sa2_b2),
               w30a, w30b, row(sa3_b0), sa3_w1, row(sa3_b1), sa3_w2, row(sa3_b2))

    def wspec(w):
        nd = w.ndim
        return pl.BlockSpec(w.shape, lambda b, _nd=nd: (0,) * _nd)

    gf = pl.pallas_call(
        _sa_kernel,
        grid=(B,),
        in_specs=[
            pl.BlockSpec((1, N, 8), lambda b: (b, 0, 0)),
            pl.BlockSpec((1, 8, N), lambda b: (b, 0, 0)),
        ] + [wspec(w) for w in weights],
        out_specs=pl.BlockSpec((1, 1024), lambda b: (b, 0)),
        out_shape=jax.ShapeDtypeStruct((B, 1024), F32),
        scratch_shapes=[
            pltpu.VMEM((S1, N), F32),
            pltpu.VMEM((K1, S1, 16), F32),
            pltpu.VMEM((S2, S1), F32),
            pltpu.VMEM((K2, S2, 128), F32),
        ],
        compiler_params=pltpu.CompilerParams(
            dimension_semantics=("arbitrary",)),
    )(xyz_pad, xyzT, *weights)

    head_in = (gf, fc1_w, row(fc1_b), row(bn1_g), row(bn1_b),
               fc2_w, row(fc2_b), row(bn2_g), row(bn2_b),
               fc3_w, row(fc3_b))

    def hspec(x):
        nd = x.ndim
        return pl.BlockSpec(x.shape, lambda _nd=nd: (0,) * _nd)

    out = pl.pallas_call(
        _head_kernel,
        in_specs=[hspec(x) for x in head_in],
        out_specs=pl.BlockSpec((B, 40), lambda: (0, 0)),
        out_shape=jax.ShapeDtypeStruct((B, 40), F32),
    )(*head_in)
    return out


# speedup vs baseline: 3.7044x; 3.7044x over previous
import jax
import jax.numpy as jnp
from jax.experimental import pallas as pl
from jax.experimental.pallas import tpu as pltpu

B, N, S1, K1, S2, K2 = 16, 2048, 512, 32, 128, 64
F32 = jnp.float32


def _mm(a, b):
    return jnp.dot(a, b, preferred_element_type=F32)


def _mm_exact(a, b):
    return jnp.dot(a, b, preferred_element_type=F32,
                   precision=jax.lax.Precision.HIGHEST)


def _pairwise_d2(xyz, xyzT, n_rows, n_cols):
    acc = None
    for c in range(3):
        df = xyz[:n_rows, c:c + 1] - xyzT[c:c + 1, :n_cols]
        sq = df * df
        acc = sq if acc is None else acc + sq
    return acc


def _sa_kernel(xyz_ref, xyzT_ref,
               w0p_ref, b0_ref, w1_ref, b1_ref, w2_ref, b2_ref,
               w20a_ref, w20b_ref, b20_ref, w21_ref, b21_ref, w22_ref, b22_ref,
               w30a_ref, w30b_ref, b30_ref, w31_ref, b31_ref, w32_ref, b32_ref,
               out_ref, d2_ref, g1_ref, d22_ref, g2_ref):
    xyz = xyz_ref[0]
    xyzT = xyzT_ref[0]

    d2_ref[...] = _pairwise_d2(xyz, xyzT, S1, N)
    iota1 = jax.lax.broadcasted_iota(jnp.int32, (S1, N), 1)
    new_xyz1 = xyz[:S1, :]

    def step1(j, carry):
        d2 = d2_ref[...]
        rowmin = jnp.min(d2, axis=1, keepdims=True)
        amin = jnp.min(jnp.where(d2 == rowmin, iota1, N), axis=1, keepdims=True)
        oh_b = iota1 == amin
        oh = oh_b.astype(F32)
        sel = _mm_exact(oh, xyz)
        g1_ref[j] = jnp.concatenate([sel - new_xyz1, sel], axis=1)
        d2_ref[...] = jnp.where(oh_b, jnp.float32(jnp.inf), d2)
        return carry

    jax.lax.fori_loop(0, K1, step1, 0)

    l1f = None
    CH1 = 8
    for c0 in range(0, K1, CH1):
        x = g1_ref[c0:c0 + CH1].reshape(CH1 * S1, 16)
        h = jnp.maximum(_mm(x, w0p_ref[...]) + b0_ref[...], 0.0)
        h = jnp.maximum(_mm(h, w1_ref[...]) + b1_ref[...], 0.0)
        h = jnp.maximum(_mm(h, w2_ref[...]) + b2_ref[...], 0.0)
        for j in range(CH1):
            hj = h[j * S1:(j + 1) * S1]
            l1f = hj if l1f is None else jnp.maximum(l1f, hj)

    d22_ref[...] = _pairwise_d2(xyz, xyzT, S2, S1)
    iota2 = jax.lax.broadcasted_iota(jnp.int32, (S2, S1), 1)
    a2 = _mm(xyz[:S1, :], w20a_ref[...]) + _mm(l1f, w20b_ref[...]) + b20_ref[...]
    c2 = _mm(xyz[:S2, :], w20a_ref[...])

    def step2(j, carry):
        d2 = d22_ref[...]
        rowmin = jnp.min(d2, axis=1, keepdims=True)
        amin = jnp.min(jnp.where(d2 == rowmin, iota2, S1), axis=1, keepdims=True)
        oh_b = iota2 == amin
        oh = oh_b.astype(F32)
        sel = _mm_exact(oh, a2)
        g2_ref[j] = jnp.maximum(sel - c2, 0.0)
        d22_ref[...] = jnp.where(oh_b, jnp.float32(jnp.inf), d2)
        return carry

    jax.lax.fori_loop(0, K2, step2, 0)

    l2f = None
    CH2 = 16
    for c0 in range(0, K2, CH2):
        x = g2_ref[c0:c0 + CH2].reshape(CH2 * S2, 128)
        h = jnp.maximum(_mm(x, w21_ref[...]) + b21_ref[...], 0.0)
        h = jnp.maximum(_mm(h, w22_ref[...]) + b22_ref[...], 0.0)
        for j in range(CH2):
            hj = h[j * S2:(j + 1) * S2]
            l2f = hj if l2f is None else jnp.maximum(l2f, hj)

    h = jnp.maximum(_mm(xyz[:S2, :], w30a_ref[...]) + _mm(l2f, w30b_ref[...])
                    + b30_ref[...], 0.0)
    h = jnp.maximum(_mm(h, w31_ref[...]) + b31_ref[...], 0.0)
    h = jnp.maximum(_mm(h, w32_ref[...]) + b32_ref[...], 0.0)
    out_ref[0] = jnp.max(h, axis=0, keepdims=True)


def _head_kernel(gf_ref, fc1w_ref, fc1b_ref, bn1g_ref, bn1b_ref,
                 fc2w_ref, fc2b_ref, bn2g_ref, bn2b_ref,
                 fc3w_ref, fc3b_ref, out_ref):
    def bn_relu(x, g, b):
        m = jnp.mean(x, axis=0, keepdims=True)
        v = jnp.mean((x - m) ** 2, axis=0, keepdims=True)
        return jnp.maximum(g * (x - m) / jnp.sqrt(v + 1e-5) + b, 0.0)

    h = _mm(gf_ref[...], fc1w_ref[...]) + fc1b_ref[...]
    h = bn_relu(h, bn1g_ref[...], bn1b_ref[...])
    h = _mm(h, fc2w_ref[...]) + fc2b_ref[...]
    h = bn_relu(h, bn2g_ref[...], bn2b_ref[...])
    s = _mm(h, fc3w_ref[...]) + fc3b_ref[...]
    s = s - jnp.max(s, axis=1, keepdims=True)
    out_ref[...] = s - jnp.log(jnp.sum(jnp.exp(s), axis=1, keepdims=True))


def kernel(pts, sa1_w0, sa1_b0, sa1_w1, sa1_b1, sa1_w2, sa1_b2,
           sa2_w0, sa2_b0, sa2_w1, sa2_b1, sa2_w2, sa2_b2,
           sa3_w0, sa3_b0, sa3_w1, sa3_b1, sa3_w2, sa3_b2,
           fc1_w, fc1_b, bn1_g, bn1_b, fc2_w, fc2_b, bn2_g, bn2_b,
           fc3_w, fc3_b):
    xyz_pad = jnp.concatenate(
        [pts, jnp.zeros((B, N, 5), F32)], axis=-1)
    xyzT = jnp.transpose(xyz_pad, (0, 2, 1))

    w0p = jnp.zeros((16, 64), F32)
    w0p = w0p.at[0:3].set(sa1_w0[0:3]).at[8:11].set(sa1_w0[3:6])
    w20a = jnp.zeros((8, 128), F32).at[0:3].set(sa2_w0[0:3])
    w20b = sa2_w0[3:]
    w30a = jnp.zeros((8, 256), F32).at[0:3].set(sa3_w0[0:3])
    w30b = sa3_w0[3:]

    def row(v):
        return v[None, :]

    weights = (w0p, row(sa1_b0), sa1_w1, row(sa1_b1), sa1_w2, row(sa1_b2),
               w20a, w20b, row(sa2_b0), sa2_w1, row(sa2_b1), sa2_w2, row(sa2_b2),
               w30a, w30b, row(sa3_b0), sa3_w1, row(sa3_b1), sa3_w2, row(sa3_b2))

    def wspec(w):
        nd = w.ndim
        return pl.BlockSpec(w.shape, lambda b, _nd=nd: (0,) * _nd)

    gf = pl.pallas_call(
        _sa_kernel,
        grid=(B,),
        in_specs=[
            pl.BlockSpec((1, N, 8), lambda b: (b, 0, 0)),
            pl.BlockSpec((1, 8, N), lambda b: (b, 0, 0)),
        ] + [wspec(w) for w in weights],
        out_specs=pl.BlockSpec((1, 1, 1024), lambda b: (b, 0, 0)),
        out_shape=jax.ShapeDtypeStruct((B, 1, 1024), F32),
        scratch_shapes=[
            pltpu.VMEM((S1, N), F32),
            pltpu.VMEM((K1, S1, 16), F32),
            pltpu.VMEM((S2, S1), F32),
            pltpu.VMEM((K2, S2, 128), F32),
        ],
        compiler_params=pltpu.CompilerParams(
            dimension_semantics=("arbitrary",)),
    )(xyz_pad, xyzT, *weights)
    gf = gf.reshape(B, 1024)

    head_in = (gf, fc1_w, row(fc1_b), row(bn1_g), row(bn1_b),
               fc2_w, row(fc2_b), row(bn2_g), row(bn2_b),
               fc3_w, row(fc3_b))

    def hspec(x):
        nd = x.ndim
        return pl.BlockSpec(x.shape, lambda _nd=nd: (0,) * _nd)

    out = pl.pallas_call(
        _head_kernel,
        in_specs=[hspec(x) for x in head_in],
        out_specs=pl.BlockSpec((B, 40), lambda: (0, 0)),
        out_shape=jax.ShapeDtypeStruct((B, 40), F32),
    )(*head_in)
    return out


# speedup vs baseline: 8.6298x; 2.3296x over previous
import jax
import jax.numpy as jnp
from jax.experimental import pallas as pl
from jax.experimental.pallas import tpu as pltpu

B, N, S1, K1, S2, K2 = 16, 2048, 512, 32, 128, 64
F32 = jnp.float32


def _mm(a, b):
    return jnp.dot(a, b, preferred_element_type=F32)


def _pairwise_d2(xyz, xyzT, n_rows, n_cols):
    acc = None
    for c in range(3):
        df = xyz[:n_rows, c:c + 1] - xyzT[c:c + 1, :n_cols]
        sq = df * df
        acc = sq if acc is None else acc + sq
    return acc


def _sa_kernel(xyz_ref, xyzT_ref,
               w0s_ref, w0r_ref, b0_ref, w1_ref, b1_ref, w2_ref, b2_ref,
               w20a_ref, w20b_ref, b20_ref, w21_ref, b21_ref, w22_ref, b22_ref,
               w30a_ref, w30b_ref, b30_ref, w31_ref, b31_ref, w32_ref, b32_ref,
               out_ref, d2_ref, g1_ref, d22_ref, g2_ref):
    xyz = xyz_ref[0]
    xyzT = xyzT_ref[0]

    d2_ref[...] = _pairwise_d2(xyz, xyzT, S1, N)
    iota1 = jax.lax.broadcasted_iota(jnp.int32, (S1, N), 1)
    a1 = _mm(xyz, w0s_ref[...]) + b0_ref[...]
    c1 = _mm(xyz[:S1, :], w0r_ref[...])

    def step1(j, carry):
        d2 = d2_ref[...]
        rowmin = jnp.min(d2, axis=1, keepdims=True)
        amin = jnp.min(jnp.where(d2 == rowmin, iota1, N), axis=1, keepdims=True)
        oh_b = iota1 == amin
        g1_ref[j] = _mm(oh_b.astype(F32), a1)
        d2_ref[...] = jnp.where(oh_b, jnp.float32(jnp.inf), d2)
        return carry

    jax.lax.fori_loop(0, K1, step1, 0)

    l1f = None
    CH1 = 8
    for c0 in range(0, K1, CH1):
        x = jnp.maximum(g1_ref[c0:c0 + CH1] - c1[None], 0.0)
        x = x.reshape(CH1 * S1, 64)
        h = jnp.maximum(_mm(x, w1_ref[...]) + b1_ref[...], 0.0)
        h = jnp.maximum(_mm(h, w2_ref[...]) + b2_ref[...], 0.0)
        for j in range(CH1):
            hj = h[j * S1:(j + 1) * S1]
            l1f = hj if l1f is None else jnp.maximum(l1f, hj)

    d22_ref[...] = _pairwise_d2(xyz, xyzT, S2, S1)
    iota2 = jax.lax.broadcasted_iota(jnp.int32, (S2, S1), 1)
    a2 = _mm(xyz[:S1, :], w20a_ref[...]) + _mm(l1f, w20b_ref[...]) + b20_ref[...]
    c2 = _mm(xyz[:S2, :], w20a_ref[...])

    def step2(j, carry):
        d2 = d22_ref[...]
        rowmin = jnp.min(d2, axis=1, keepdims=True)
        amin = jnp.min(jnp.where(d2 == rowmin, iota2, S1), axis=1, keepdims=True)
        oh_b = iota2 == amin
        oh = oh_b.astype(F32)
        sel = _mm(oh, a2)
        g2_ref[j] = jnp.maximum(sel - c2, 0.0)
        d22_ref[...] = jnp.where(oh_b, jnp.float32(jnp.inf), d2)
        return carry

    jax.lax.fori_loop(0, K2, step2, 0)

    l2f = None
    CH2 = 16
    for c0 in range(0, K2, CH2):
        x = g2_ref[c0:c0 + CH2].reshape(CH2 * S2, 128)
        h = jnp.maximum(_mm(x, w21_ref[...]) + b21_ref[...], 0.0)
        h = jnp.maximum(_mm(h, w22_ref[...]) + b22_ref[...], 0.0)
        for j in range(CH2):
            hj = h[j * S2:(j + 1) * S2]
            l2f = hj if l2f is None else jnp.maximum(l2f, hj)

    h = jnp.maximum(_mm(xyz[:S2, :], w30a_ref[...]) + _mm(l2f, w30b_ref[...])
                    + b30_ref[...], 0.0)
    h = jnp.maximum(_mm(h, w31_ref[...]) + b31_ref[...], 0.0)
    h = jnp.maximum(_mm(h, w32_ref[...]) + b32_ref[...], 0.0)
    out_ref[0] = jnp.max(h, axis=0, keepdims=True)


def _head_kernel(gf_ref, fc1w_ref, fc1b_ref, bn1g_ref, bn1b_ref,
                 fc2w_ref, fc2b_ref, bn2g_ref, bn2b_ref,
                 fc3w_ref, fc3b_ref, out_ref):
    def bn_relu(x, g, b):
        m = jnp.mean(x, axis=0, keepdims=True)
        v = jnp.mean((x - m) ** 2, axis=0, keepdims=True)
        return jnp.maximum(g * (x - m) / jnp.sqrt(v + 1e-5) + b, 0.0)

    h = _mm(gf_ref[...], fc1w_ref[...]) + fc1b_ref[...]
    h = bn_relu(h, bn1g_ref[...], bn1b_ref[...])
    h = _mm(h, fc2w_ref[...]) + fc2b_ref[...]
    h = bn_relu(h, bn2g_ref[...], bn2b_ref[...])
    s = _mm(h, fc3w_ref[...]) + fc3b_ref[...]
    s = s - jnp.max(s, axis=1, keepdims=True)
    out_ref[...] = s - jnp.log(jnp.sum(jnp.exp(s), axis=1, keepdims=True))


def kernel(pts, sa1_w0, sa1_b0, sa1_w1, sa1_b1, sa1_w2, sa1_b2,
           sa2_w0, sa2_b0, sa2_w1, sa2_b1, sa2_w2, sa2_b2,
           sa3_w0, sa3_b0, sa3_w1, sa3_b1, sa3_w2, sa3_b2,
           fc1_w, fc1_b, bn1_g, bn1_b, fc2_w, fc2_b, bn2_g, bn2_b,
           fc3_w, fc3_b):
    xyz_pad = jnp.concatenate(
        [pts, jnp.zeros((B, N, 5), F32)], axis=-1)
    xyzT = jnp.transpose(xyz_pad, (0, 2, 1))

    w0s = jnp.zeros((8, 64), F32).at[0:3].set(sa1_w0[0:3] + sa1_w0[3:6])
    w0r = jnp.zeros((8, 64), F32).at[0:3].set(sa1_w0[0:3])
    w20a = jnp.zeros((8, 128), F32).at[0:3].set(sa2_w0[0:3])
    w20b = sa2_w0[3:]
    w30a = jnp.zeros((8, 256), F32).at[0:3].set(sa3_w0[0:3])
    w30b = sa3_w0[3:]

    def row(v):
        return v[None, :]

    weights = (w0s, w0r, row(sa1_b0), sa1_w1, row(sa1_b1), sa1_w2, row(sa1_b2),
               w20a, w20b, row(sa2_b0), sa2_w1, row(sa2_b1), sa2_w2, row(sa2_b2),
               w30a, w30b, row(sa3_b0), sa3_w1, row(sa3_b1), sa3_w2, row(sa3_b2))

    def wspec(w):
        nd = w.ndim
        return pl.BlockSpec(w.shape, lambda b, _nd=nd: (0,) * _nd)

    gf = pl.pallas_call(
        _sa_kernel,
        grid=(B,),
        in_specs=[
            pl.BlockSpec((1, N, 8), lambda b: (b, 0, 0)),
            pl.BlockSpec((1, 8, N), lambda b: (b, 0, 0)),
        ] + [wspec(w) for w in weights],
        out_specs=pl.BlockSpec((1, 1, 1024), lambda b: (b, 0, 0)),
        out_shape=jax.ShapeDtypeStruct((B, 1, 1024), F32),
        scratch_shapes=[
            pltpu.VMEM((S1, N), F32),
            pltpu.VMEM((K1, S1, 64), F32),
            pltpu.VMEM((S2, S1), F32),
            pltpu.VMEM((K2, S2, 128), F32),
        ],
        compiler_params=pltpu.CompilerParams(
            dimension_semantics=("arbitrary",)),
    )(xyz_pad, xyzT, *weights)
    gf = gf.reshape(B, 1024)

    head_in = (gf, fc1_w, row(fc1_b), row(bn1_g), row(bn1_b),
               fc2_w, row(fc2_b), row(bn2_g), row(bn2_b),
               fc3_w, row(fc3_b))

    def hspec(x):
        nd = x.ndim
        return pl.BlockSpec(x.shape, lambda _nd=nd: (0,) * _nd)

    out = pl.pallas_call(
        _head_kernel,
        in_specs=[hspec(x) for x in head_in],
        out_specs=pl.BlockSpec((B, 40), lambda: (0, 0)),
        out_shape=jax.ShapeDtypeStruct((B, 40), F32),
    )(*head_in)
    return out


# speedup vs baseline: 10.7332x; 1.2437x over previous
import jax
import jax.numpy as jnp
from jax.experimental import pallas as pl
from jax.experimental.pallas import tpu as pltpu

B, N, S1, K1, S2, K2 = 16, 2048, 512, 32, 128, 64
F32 = jnp.float32


def _mm(a, b):
    return jnp.dot(a, b, preferred_element_type=F32)


def _pairwise_d2(xyz, xyzT, n_rows, n_cols):
    acc = None
    for c in range(3):
        df = xyz[:n_rows, c:c + 1] - xyzT[c:c + 1, :n_cols]
        sq = df * df
        acc = sq if acc is None else acc + sq
    return acc


def _sa_kernel(xyz_ref, xyzT_ref,
               w0s_ref, w0r_ref, b0_ref, w1_ref, b1_ref, w2_ref, b2_ref,
               w20a_ref, w20b_ref, b20_ref, w21_ref, b21_ref, w22_ref, b22_ref,
               w30a_ref, w30b_ref, b30_ref, w31_ref, b31_ref, w32_ref, b32_ref,
               out_ref, d2_ref, g1_ref, d22_ref, g2_ref):
    xyz = xyz_ref[0]
    xyzT = xyzT_ref[0]

    d2_ref[...] = _pairwise_d2(xyz, xyzT, S1, N)
    iota1 = jax.lax.broadcasted_iota(jnp.int32, (S1, N), 1)
    a1 = _mm(xyz, w0s_ref[...]) + b0_ref[...]
    c1 = _mm(xyz[:S1, :], w0r_ref[...])

    for j in range(K1):
        d2 = d2_ref[...]
        rowmin = jnp.min(d2, axis=1, keepdims=True)
        amin = jnp.min(jnp.where(d2 == rowmin, iota1, N), axis=1, keepdims=True)
        oh_b = iota1 == amin
        g1_ref[j] = _mm(oh_b.astype(F32), a1)
        d2_ref[...] = jnp.where(oh_b, jnp.float32(jnp.inf), d2)

    l1f = None
    CH1 = 8
    for c0 in range(0, K1, CH1):
        x = jnp.maximum(g1_ref[c0:c0 + CH1] - c1[None], 0.0)
        x = x.reshape(CH1 * S1, 64)
        h = jnp.maximum(_mm(x, w1_ref[...]) + b1_ref[...], 0.0)
        h = jnp.maximum(_mm(h, w2_ref[...]) + b2_ref[...], 0.0)
        for j in range(CH1):
            hj = h[j * S1:(j + 1) * S1]
            l1f = hj if l1f is None else jnp.maximum(l1f, hj)

    d22_ref[...] = _pairwise_d2(xyz, xyzT, S2, S1)
    iota2 = jax.lax.broadcasted_iota(jnp.int32, (S2, S1), 1)
    a2 = _mm(xyz[:S1, :], w20a_ref[...]) + _mm(l1f, w20b_ref[...]) + b20_ref[...]
    c2 = _mm(xyz[:S2, :], w20a_ref[...])

    for j in range(K2):
        d2 = d22_ref[...]
        rowmin = jnp.min(d2, axis=1, keepdims=True)
        amin = jnp.min(jnp.where(d2 == rowmin, iota2, S1), axis=1, keepdims=True)
        oh_b = iota2 == amin
        sel = _mm(oh_b.astype(F32), a2)
        g2_ref[j] = jnp.maximum(sel - c2, 0.0)
        d22_ref[...] = jnp.where(oh_b, jnp.float32(jnp.inf), d2)

    l2f = None
    CH2 = 16
    for c0 in range(0, K2, CH2):
        x = g2_ref[c0:c0 + CH2].reshape(CH2 * S2, 128)
        h = jnp.maximum(_mm(x, w21_ref[...]) + b21_ref[...], 0.0)
        h = jnp.maximum(_mm(h, w22_ref[...]) + b22_ref[...], 0.0)
        for j in range(CH2):
            hj = h[j * S2:(j + 1) * S2]
            l2f = hj if l2f is None else jnp.maximum(l2f, hj)

    h = jnp.maximum(_mm(xyz[:S2, :], w30a_ref[...]) + _mm(l2f, w30b_ref[...])
                    + b30_ref[...], 0.0)
    h = jnp.maximum(_mm(h, w31_ref[...]) + b31_ref[...], 0.0)
    h = jnp.maximum(_mm(h, w32_ref[...]) + b32_ref[...], 0.0)
    out_ref[0] = jnp.max(h, axis=0, keepdims=True)


def _head_kernel(gf_ref, fc1w_ref, fc1b_ref, bn1g_ref, bn1b_ref,
                 fc2w_ref, fc2b_ref, bn2g_ref, bn2b_ref,
                 fc3w_ref, fc3b_ref, out_ref):
    def bn_relu(x, g, b):
        m = jnp.mean(x, axis=0, keepdims=True)
        v = jnp.mean((x - m) ** 2, axis=0, keepdims=True)
        return jnp.maximum(g * (x - m) / jnp.sqrt(v + 1e-5) + b, 0.0)

    h = _mm(gf_ref[...], fc1w_ref[...]) + fc1b_ref[...]
    h = bn_relu(h, bn1g_ref[...], bn1b_ref[...])
    h = _mm(h, fc2w_ref[...]) + fc2b_ref[...]
    h = bn_relu(h, bn2g_ref[...], bn2b_ref[...])
    s = _mm(h, fc3w_ref[...]) + fc3b_ref[...]
    s = s - jnp.max(s, axis=1, keepdims=True)
    out_ref[...] = s - jnp.log(jnp.sum(jnp.exp(s), axis=1, keepdims=True))


def kernel(pts, sa1_w0, sa1_b0, sa1_w1, sa1_b1, sa1_w2, sa1_b2,
           sa2_w0, sa2_b0, sa2_w1, sa2_b1, sa2_w2, sa2_b2,
           sa3_w0, sa3_b0, sa3_w1, sa3_b1, sa3_w2, sa3_b2,
           fc1_w, fc1_b, bn1_g, bn1_b, fc2_w, fc2_b, bn2_g, bn2_b,
           fc3_w, fc3_b):
    xyz_pad = jnp.concatenate(
        [pts, jnp.zeros((B, N, 5), F32)], axis=-1)
    xyzT = jnp.transpose(xyz_pad, (0, 2, 1))

    w0s = jnp.zeros((8, 64), F32).at[0:3].set(sa1_w0[0:3] + sa1_w0[3:6])
    w0r = jnp.zeros((8, 64), F32).at[0:3].set(sa1_w0[0:3])
    w20a = jnp.zeros((8, 128), F32).at[0:3].set(sa2_w0[0:3])
    w20b = sa2_w0[3:]
    w30a = jnp.zeros((8, 256), F32).at[0:3].set(sa3_w0[0:3])
    w30b = sa3_w0[3:]

    def row(v):
        return v[None, :]

    weights = (w0s, w0r, row(sa1_b0), sa1_w1, row(sa1_b1), sa1_w2, row(sa1_b2),
               w20a, w20b, row(sa2_b0), sa2_w1, row(sa2_b1), sa2_w2, row(sa2_b2),
               w30a, w30b, row(sa3_b0), sa3_w1, row(sa3_b1), sa3_w2, row(sa3_b2))

    def wspec(w):
        nd = w.ndim
        return pl.BlockSpec(w.shape, lambda b, _nd=nd: (0,) * _nd)

    gf = pl.pallas_call(
        _sa_kernel,
        grid=(B,),
        in_specs=[
            pl.BlockSpec((1, N, 8), lambda b: (b, 0, 0)),
            pl.BlockSpec((1, 8, N), lambda b: (b, 0, 0)),
        ] + [wspec(w) for w in weights],
        out_specs=pl.BlockSpec((1, 1, 1024), lambda b: (b, 0, 0)),
        out_shape=jax.ShapeDtypeStruct((B, 1, 1024), F32),
        scratch_shapes=[
            pltpu.VMEM((S1, N), F32),
            pltpu.VMEM((K1, S1, 64), F32),
            pltpu.VMEM((S2, S1), F32),
            pltpu.VMEM((K2, S2, 128), F32),
        ],
        compiler_params=pltpu.CompilerParams(
            dimension_semantics=("arbitrary",)),
    )(xyz_pad, xyzT, *weights)
    gf = gf.reshape(B, 1024)

    head_in = (gf, fc1_w, row(fc1_b), row(bn1_g), row(bn1_b),
               fc2_w, row(fc2_b), row(bn2_g), row(bn2_b),
               fc3_w, row(fc3_b))

    def hspec(x):
        nd = x.ndim
        return pl.BlockSpec(x.shape, lambda _nd=nd: (0,) * _nd)

    out = pl.pallas_call(
        _head_kernel,
        in_specs=[hspec(x) for x in head_in],
        out_specs=pl.BlockSpec((B, 40), lambda: (0, 0)),
        out_shape=jax.ShapeDtypeStruct((B, 40), F32),
    )(*head_in)
    return out


# speedup vs baseline: 12.0641x; 1.1240x over previous
import jax
import jax.numpy as jnp
from jax.experimental import pallas as pl
from jax.experimental.pallas import tpu as pltpu

B, N, S1, K1, S2, K2 = 16, 2048, 512, 32, 128, 64
F32 = jnp.float32


def _mm(a, b):
    return jnp.dot(a, b, preferred_element_type=F32)


def _pairwise_d2(xyz, xyzT, n_rows, n_cols):
    acc = None
    for c in range(3):
        df = xyz[:n_rows, c:c + 1] - xyzT[c:c + 1, :n_cols]
        sq = df * df
        acc = sq if acc is None else acc + sq
    return acc


def _sa_kernel(xyz_ref, xyzT_ref,
               w0s_ref, w0r_ref, b0_ref, w1_ref, b1_ref, w2_ref, b2_ref,
               w20a_ref, w20b_ref, b20_ref, w21_ref, b21_ref, w22_ref, b22_ref,
               w30a_ref, w30b_ref, b30_ref, w31_ref, b31_ref, w32_ref, b32_ref,
               out_ref, d2_ref, g1_ref, d22_ref, g2_ref):
    xyz = xyz_ref[0]
    xyzT = xyzT_ref[0]

    d2_ref[...] = _pairwise_d2(xyz, xyzT, S1, N)
    iota1 = jax.lax.broadcasted_iota(jnp.int32, (S1, N), 1)
    a1 = _mm(xyz, w0s_ref[...]) + b0_ref[...]
    c1 = _mm(xyz[:S1, :], w0r_ref[...])

    for j in range(K1):
        d2 = d2_ref[...]
        amin = jnp.argmin(d2, axis=1)[:, None]
        oh_b = iota1 == amin
        g1_ref[j] = _mm(oh_b.astype(F32), a1)
        d2_ref[...] = jnp.where(oh_b, jnp.float32(jnp.inf), d2)

    l1f = None
    CH1 = 8
    for c0 in range(0, K1, CH1):
        x = jnp.maximum(g1_ref[c0:c0 + CH1] - c1[None], 0.0)
        x = x.reshape(CH1 * S1, 64)
        h = jnp.maximum(_mm(x, w1_ref[...]) + b1_ref[...], 0.0)
        h = jnp.maximum(_mm(h, w2_ref[...]) + b2_ref[...], 0.0)
        for j in range(CH1):
            hj = h[j * S1:(j + 1) * S1]
            l1f = hj if l1f is None else jnp.maximum(l1f, hj)

    d22_ref[...] = _pairwise_d2(xyz, xyzT, S2, S1)
    iota2 = jax.lax.broadcasted_iota(jnp.int32, (S2, S1), 1)
    a2 = _mm(xyz[:S1, :], w20a_ref[...]) + _mm(l1f, w20b_ref[...]) + b20_ref[...]
    c2 = _mm(xyz[:S2, :], w20a_ref[...])

    for j in range(K2):
        d2 = d22_ref[...]
        amin = jnp.argmin(d2, axis=1)[:, None]
        oh_b = iota2 == amin
        sel = _mm(oh_b.astype(F32), a2)
        g2_ref[j] = jnp.maximum(sel - c2, 0.0)
        d22_ref[...] = jnp.where(oh_b, jnp.float32(jnp.inf), d2)

    l2f = None
    CH2 = 16
    for c0 in range(0, K2, CH2):
        x = g2_ref[c0:c0 + CH2].reshape(CH2 * S2, 128)
        h = jnp.maximum(_mm(x, w21_ref[...]) + b21_ref[...], 0.0)
        h = jnp.maximum(_mm(h, w22_ref[...]) + b22_ref[...], 0.0)
        for j in range(CH2):
            hj = h[j * S2:(j + 1) * S2]
            l2f = hj if l2f is None else jnp.maximum(l2f, hj)

    h = jnp.maximum(_mm(xyz[:S2, :], w30a_ref[...]) + _mm(l2f, w30b_ref[...])
                    + b30_ref[...], 0.0)
    h = jnp.maximum(_mm(h, w31_ref[...]) + b31_ref[...], 0.0)
    h = jnp.maximum(_mm(h, w32_ref[...]) + b32_ref[...], 0.0)
    out_ref[0] = jnp.max(h, axis=0, keepdims=True)


def _head_kernel(gf_ref, fc1w_ref, fc1b_ref, bn1g_ref, bn1b_ref,
                 fc2w_ref, fc2b_ref, bn2g_ref, bn2b_ref,
                 fc3w_ref, fc3b_ref, out_ref):
    def bn_relu(x, g, b):
        m = jnp.mean(x, axis=0, keepdims=True)
        v = jnp.mean((x - m) ** 2, axis=0, keepdims=True)
        return jnp.maximum(g * (x - m) / jnp.sqrt(v + 1e-5) + b, 0.0)

    h = _mm(gf_ref[...], fc1w_ref[...]) + fc1b_ref[...]
    h = bn_relu(h, bn1g_ref[...], bn1b_ref[...])
    h = _mm(h, fc2w_ref[...]) + fc2b_ref[...]
    h = bn_relu(h, bn2g_ref[...], bn2b_ref[...])
    s = _mm(h, fc3w_ref[...]) + fc3b_ref[...]
    s = s - jnp.max(s, axis=1, keepdims=True)
    out_ref[...] = s - jnp.log(jnp.sum(jnp.exp(s), axis=1, keepdims=True))


def kernel(pts, sa1_w0, sa1_b0, sa1_w1, sa1_b1, sa1_w2, sa1_b2,
           sa2_w0, sa2_b0, sa2_w1, sa2_b1, sa2_w2, sa2_b2,
           sa3_w0, sa3_b0, sa3_w1, sa3_b1, sa3_w2, sa3_b2,
           fc1_w, fc1_b, bn1_g, bn1_b, fc2_w, fc2_b, bn2_g, bn2_b,
           fc3_w, fc3_b):
    xyz_pad = jnp.concatenate(
        [pts, jnp.zeros((B, N, 5), F32)], axis=-1)
    xyzT = jnp.transpose(xyz_pad, (0, 2, 1))

    w0s = jnp.zeros((8, 64), F32).at[0:3].set(sa1_w0[0:3] + sa1_w0[3:6])
    w0r = jnp.zeros((8, 64), F32).at[0:3].set(sa1_w0[0:3])
    w20a = jnp.zeros((8, 128), F32).at[0:3].set(sa2_w0[0:3])
    w20b = sa2_w0[3:]
    w30a = jnp.zeros((8, 256), F32).at[0:3].set(sa3_w0[0:3])
    w30b = sa3_w0[3:]

    def row(v):
        return v[None, :]

    weights = (w0s, w0r, row(sa1_b0), sa1_w1, row(sa1_b1), sa1_w2, row(sa1_b2),
               w20a, w20b, row(sa2_b0), sa2_w1, row(sa2_b1), sa2_w2, row(sa2_b2),
               w30a, w30b, row(sa3_b0), sa3_w1, row(sa3_b1), sa3_w2, row(sa3_b2))

    def wspec(w):
        nd = w.ndim
        return pl.BlockSpec(w.shape, lambda b, _nd=nd: (0,) * _nd)

    gf = pl.pallas_call(
        _sa_kernel,
        grid=(B,),
        in_specs=[
            pl.BlockSpec((1, N, 8), lambda b: (b, 0, 0)),
            pl.BlockSpec((1, 8, N), lambda b: (b, 0, 0)),
        ] + [wspec(w) for w in weights],
        out_specs=pl.BlockSpec((1, 1, 1024), lambda b: (b, 0, 0)),
        out_shape=jax.ShapeDtypeStruct((B, 1, 1024), F32),
        scratch_shapes=[
            pltpu.VMEM((S1, N), F32),
            pltpu.VMEM((K1, S1, 64), F32),
            pltpu.VMEM((S2, S1), F32),
            pltpu.VMEM((K2, S2, 128), F32),
        ],
        compiler_params=pltpu.CompilerParams(
            dimension_semantics=("arbitrary",)),
    )(xyz_pad, xyzT, *weights)
    gf = gf.reshape(B, 1024)

    head_in = (gf, fc1_w, row(fc1_b), row(bn1_g), row(bn1_b),
               fc2_w, row(fc2_b), row(bn2_g), row(bn2_b),
               fc3_w, row(fc3_b))

    def hspec(x):
        nd = x.ndim
        return pl.BlockSpec(x.shape, lambda _nd=nd: (0,) * _nd)

    out = pl.pallas_call(
        _head_kernel,
        in_specs=[hspec(x) for x in head_in],
        out_specs=pl.BlockSpec((B, 40), lambda: (0, 0)),
        out_shape=jax.ShapeDtypeStruct((B, 40), F32),
    )(*head_in)
    return out
